# baseline (speedup 1.0000x reference)
import jax
import jax.numpy as jnp
from jax.experimental import pallas as pl
from jax.experimental.pallas import tpu as pltpu

_B, _L, _D, _A, _ACT = 16, 1024, 256, 128, 1024
_H1, _H2 = 512, 1024


def _attn_mlp_body(state_ref, now_ref, ne_ref, wa_ref, ba_ref, ww_ref, bw_ref,
                   w1_ref, b1_ref, w2_ref, b2_ref, w3_ref, b3_ref,
                   out_ref, agg_ref):
    b = pl.program_id(0)
    ne = ne_ref[0]
    wa = wa_ref[...]
    ww_top = ww_ref[0:_A, :]
    ww_bot = ww_ref[_A:2 * _A, :]

    v = jnp.dot(wa, ww_bot, preferred_element_type=jnp.float32)
    u = jnp.dot(wa, ww_top, preferred_element_type=jnp.float32)

    now_row = now_ref[pl.ds(b, 1), :]
    c = jnp.dot(now_row, u, preferred_element_type=jnp.float32)
    c = c + jnp.dot(ba_ref[...], ww_top, preferred_element_type=jnp.float32)
    c = c + jnp.dot(ba_ref[...], ww_bot, preferred_element_type=jnp.float32)
    c = c + bw_ref[...]

    scores = jnp.dot(ne, v, preferred_element_type=jnp.float32) + c
    scores = jnp.where(scores >= 0.0, scores, 0.2 * scores)
    m = jnp.max(scores)
    e = jnp.exp(scores - m)
    alpha = e / jnp.sum(e)

    agg_row = jnp.sum(alpha * ne, axis=0, keepdims=True)
    agg_ref[pl.ds(b, 1), :] = agg_row

    @pl.when(b == _B - 1)
    def _mlp():
        st = state_ref[...]
        ag = agg_ref[...]
        h = jnp.dot(st, w1_ref[0:2 * _D, :], preferred_element_type=jnp.float32)
        h = h + jnp.dot(ag, w1_ref[2 * _D:3 * _D, :], preferred_element_type=jnp.float32)
        h = jax.nn.relu(h + b1_ref[...])
        h = jax.nn.relu(jnp.dot(h, w2_ref[...], preferred_element_type=jnp.float32) + b2_ref[...])
        logits = jnp.dot(h, w3_ref[...], preferred_element_type=jnp.float32) + b3_ref[...]
        z = logits - jnp.max(logits, axis=1, keepdims=True)
        ez = jnp.exp(z)
        out_ref[...] = ez / jnp.sum(ez, axis=1, keepdims=True)


def kernel(state, now_embedding, neighbour_embeddings, Wa, ba, Ww, bw, W1, b1, W2, b2, W3, b3):
    ba2 = ba.reshape(1, _A)
    bw2 = bw.reshape(1, 1)
    b12 = b1.reshape(1, _H1)
    b22 = b2.reshape(1, _H2)
    b32 = b3.reshape(1, _ACT)

    const = lambda *_: (0, 0)
    grid = (_B,)
    in_specs = [
            pl.BlockSpec((_B, 2 * _D), const),
            pl.BlockSpec((_B, _D), const),
            pl.BlockSpec((1, _L, _D), lambda b: (b, 0, 0)),
            pl.BlockSpec((_D, _A), const),
            pl.BlockSpec((1, _A), const),
            pl.BlockSpec((2 * _A, 1), const),
            pl.BlockSpec((1, 1), const),
            pl.BlockSpec((3 * _D, _H1), const),
            pl.BlockSpec((1, _H1), const),
            pl.BlockSpec((_H1, _H2), const),
            pl.BlockSpec((1, _H2), const),
            pl.BlockSpec((_H2, _ACT), const),
            pl.BlockSpec((1, _ACT), const),
    ]
    return pl.pallas_call(
        _attn_mlp_body,
        grid=grid,
        in_specs=in_specs,
        out_specs=pl.BlockSpec((_B, _ACT), const),
        out_shape=jax.ShapeDtypeStruct((_B, _ACT), jnp.float32),
        scratch_shapes=[pltpu.VMEM((_B, _D), jnp.float32)],
    )(state, now_embedding, neighbour_embeddings, Wa, ba2, Ww, bw2,
      W1, b12, W2, b22, W3, b32)


# speedup vs baseline: 1.1365x; 1.1365x over previous
import jax
import jax.numpy as jnp
from jax.experimental import pallas as pl
from jax.experimental.pallas import tpu as pltpu

_B, _L, _D, _A, _ACT = 16, 1024, 256, 128, 1024
_H1, _H2 = 512, 1024


def _attn_mlp_body(state_ref, now_ref, ne_ref, wa_ref, ba_ref, ww_ref, bw_ref,
                   w1_ref, b1_ref, w2_ref, b2_ref, w3_ref, b3_ref,
                   out_ref, agg_ref, v_scr, c_scr):
    b = pl.program_id(0)
    ne = ne_ref[0]

    @pl.when(b == 0)
    def _prep():
        wa = wa_ref[...]
        ww_top = ww_ref[0:_A, :]
        ww_bot = ww_ref[_A:2 * _A, :]
        v_scr[...] = jax.lax.dot_general(
            ww_bot, wa, (((0,), (1,)), ((), ())),
            preferred_element_type=jnp.float32)
        u = jnp.dot(wa, ww_top, preferred_element_type=jnp.float32)
        c_all = jnp.dot(now_ref[...], u, preferred_element_type=jnp.float32)
        c_all = c_all + jnp.dot(ba_ref[...], ww_top, preferred_element_type=jnp.float32)
        c_all = c_all + jnp.dot(ba_ref[...], ww_bot, preferred_element_type=jnp.float32)
        c_scr[...] = c_all + bw_ref[...]

    scores = jax.lax.dot_general(
        v_scr[...], ne, (((1,), (1,)), ((), ())),
        preferred_element_type=jnp.float32)
    scores = scores + c_scr[pl.ds(b, 1), :]
    scores = jnp.where(scores >= 0.0, scores, 0.2 * scores)
    m = jnp.max(scores, axis=1, keepdims=True)
    e = jnp.exp(scores - m)
    alpha = e / jnp.sum(e, axis=1, keepdims=True)

    agg_ref[pl.ds(b, 1), :] = jnp.dot(alpha, ne,
                                      preferred_element_type=jnp.float32)

    @pl.when(b == _B - 1)
    def _mlp():
        st = state_ref[...]
        ag = agg_ref[...]
        h = jnp.dot(st, w1_ref[0:2 * _D, :], preferred_element_type=jnp.float32)
        h = h + jnp.dot(ag, w1_ref[2 * _D:3 * _D, :], preferred_element_type=jnp.float32)
        h = jax.nn.relu(h + b1_ref[...])
        h = jax.nn.relu(jnp.dot(h, w2_ref[...], preferred_element_type=jnp.float32) + b2_ref[...])
        logits = jnp.dot(h, w3_ref[...], preferred_element_type=jnp.float32) + b3_ref[...]
        z = logits - jnp.max(logits, axis=1, keepdims=True)
        ez = jnp.exp(z)
        out_ref[...] = ez / jnp.sum(ez, axis=1, keepdims=True)


def kernel(state, now_embedding, neighbour_embeddings, Wa, ba, Ww, bw, W1, b1, W2, b2, W3, b3):
    ba2 = ba.reshape(1, _A)
    bw2 = bw.reshape(1, 1)
    b12 = b1.reshape(1, _H1)
    b22 = b2.reshape(1, _H2)
    b32 = b3.reshape(1, _ACT)

    const = lambda *_: (0, 0)
    grid = (_B,)
    in_specs = [
            pl.BlockSpec((_B, 2 * _D), const),
            pl.BlockSpec((_B, _D), const),
            pl.BlockSpec((1, _L, _D), lambda b: (b, 0, 0)),
            pl.BlockSpec((_D, _A), const),
            pl.BlockSpec((1, _A), const),
            pl.BlockSpec((2 * _A, 1), const),
            pl.BlockSpec((1, 1), const),
            pl.BlockSpec((3 * _D, _H1), const),
            pl.BlockSpec((1, _H1), const),
            pl.BlockSpec((_H1, _H2), const),
            pl.BlockSpec((1, _H2), const),
            pl.BlockSpec((_H2, _ACT), const),
            pl.BlockSpec((1, _ACT), const),
    ]
    return pl.pallas_call(
        _attn_mlp_body,
        grid=grid,
        in_specs=in_specs,
        out_specs=pl.BlockSpec((_B, _ACT), const),
        out_shape=jax.ShapeDtypeStruct((_B, _ACT), jnp.float32),
        scratch_shapes=[pltpu.VMEM((_B, _D), jnp.float32),
                        pltpu.VMEM((1, _D), jnp.float32),
                        pltpu.VMEM((_B, 1), jnp.float32)],
    )(state, now_embedding, neighbour_embeddings, Wa, ba2, Ww, bw2,
      W1, b12, W2, b22, W3, b32)


# speedup vs baseline: 1.8139x; 1.5960x over previous
import jax
import jax.numpy as jnp
from jax.experimental import pallas as pl
from jax.experimental.pallas import tpu as pltpu

_B, _L, _D, _A, _ACT = 16, 1024, 256, 128, 1024
_H1, _H2 = 512, 1024
_R = 4
_STEPS = _B // _R


def _attn_mlp_body(state_ref, now_ref, ne_ref, wa_ref, ba_ref, ww_ref, bw_ref,
                   w1_ref, b1_ref, w2_ref, b2_ref, w3_ref, b3_ref,
                   out_ref, agg_ref, v_scr, c_scr):
    b = pl.program_id(0)
    ne = ne_ref[...]

    @pl.when(b == 0)
    def _prep():
        wa = wa_ref[...]
        ww_top = ww_ref[0:_A, :]
        ww_bot = ww_ref[_A:2 * _A, :]
        v_scr[...] = jax.lax.dot_general(
            ww_bot, wa, (((0,), (1,)), ((), ())),
            preferred_element_type=jnp.float32)
        u = jnp.dot(wa, ww_top, preferred_element_type=jnp.float32)
        c_all = jnp.dot(now_ref[...], u, preferred_element_type=jnp.float32)
        c_all = c_all + jnp.dot(ba_ref[...], ww_top, preferred_element_type=jnp.float32)
        c_all = c_all + jnp.dot(ba_ref[...], ww_bot, preferred_element_type=jnp.float32)
        c_all = c_all + bw_ref[...]
        for s in range(_STEPS):
            c_scr[pl.ds(s * 8, _R), :] = c_all[s * _R:(s + 1) * _R, :]

    scores = jax.lax.dot_general(
        v_scr[...], ne, (((1,), (1,)), ((), ())),
        preferred_element_type=jnp.float32)
    c_blk = c_scr[pl.ds(pl.multiple_of(b * 8, 8), _R), :]

    aggs = []
    for r in range(_R):
        s = scores[:, r * _L:(r + 1) * _L] + c_blk[r:r + 1, :]
        s = jnp.where(s >= 0.0, s, 0.2 * s)
        m = jnp.max(s, axis=1, keepdims=True)
        e = jnp.exp(s - m)
        alpha = e / jnp.sum(e, axis=1, keepdims=True)
        aggs.append(jnp.dot(alpha, ne[r * _L:(r + 1) * _L, :],
                            preferred_element_type=jnp.float32))
    agg_ref[pl.ds(pl.multiple_of(b * 8, 8), _R), :] = jnp.concatenate(aggs, axis=0)

    @pl.when(b == _STEPS - 1)
    def _mlp():
        st = state_ref[...]
        ag = jnp.concatenate(
            [agg_ref[pl.ds(s * 8, _R), :] for s in range(_STEPS)], axis=0)
        h = jnp.dot(st, w1_ref[0:2 * _D, :], preferred_element_type=jnp.float32)
        h = h + jnp.dot(ag, w1_ref[2 * _D:3 * _D, :], preferred_element_type=jnp.float32)
        h = jax.nn.relu(h + b1_ref[...])
        h = jax.nn.relu(jnp.dot(h, w2_ref[...], preferred_element_type=jnp.float32) + b2_ref[...])
        logits = jnp.dot(h, w3_ref[...], preferred_element_type=jnp.float32) + b3_ref[...]
        z = logits - jnp.max(logits, axis=1, keepdims=True)
        ez = jnp.exp(z)
        out_ref[...] = ez / jnp.sum(ez, axis=1, keepdims=True)


def kernel(state, now_embedding, neighbour_embeddings, Wa, ba, Ww, bw, W1, b1, W2, b2, W3, b3):
    ne_flat = neighbour_embeddings.reshape(_B * _L, _D)
    ba2 = ba.reshape(1, _A)
    bw2 = bw.reshape(1, 1)
    b12 = b1.reshape(1, _H1)
    b22 = b2.reshape(1, _H2)
    b32 = b3.reshape(1, _ACT)

    const = lambda *_: (0, 0)
    in_specs = [
            pl.BlockSpec((_B, 2 * _D), const),
            pl.BlockSpec((_B, _D), const),
            pl.BlockSpec((_R * _L, _D), lambda b: (b, 0)),
            pl.BlockSpec((_D, _A), const),
            pl.BlockSpec((1, _A), const),
            pl.BlockSpec((2 * _A, 1), const),
            pl.BlockSpec((1, 1), const),
            pl.BlockSpec((3 * _D, _H1), const),
            pl.BlockSpec((1, _H1), const),
            pl.BlockSpec((_H1, _H2), const),
            pl.BlockSpec((1, _H2), const),
            pl.BlockSpec((_H2, _ACT), const),
            pl.BlockSpec((1, _ACT), const),
    ]
    return pl.pallas_call(
        _attn_mlp_body,
        grid=(_STEPS,),
        in_specs=in_specs,
        out_specs=pl.BlockSpec((_B, _ACT), const),
        out_shape=jax.ShapeDtypeStruct((_B, _ACT), jnp.float32),
        scratch_shapes=[pltpu.VMEM((_STEPS * 8, _D), jnp.float32),
                        pltpu.VMEM((1, _D), jnp.float32),
                        pltpu.VMEM((_STEPS * 8, 1), jnp.float32)],
    )(state, now_embedding, ne_flat, Wa, ba2, Ww, bw2,
      W1, b12, W2, b22, W3, b32)


# speedup vs baseline: 1.8570x; 1.0238x over previous
import jax
import jax.numpy as jnp
from jax.experimental import pallas as pl
from jax.experimental.pallas import tpu as pltpu

_B, _L, _D, _A, _ACT = 16, 1024, 256, 128, 1024
_H1, _H2 = 512, 1024
_R = 8
_STEPS = _B // _R


def _attn_mlp_body(state_ref, now_ref, ne_ref, wa_ref, ba_ref, ww_ref, bw_ref,
                   w1_ref, b1_ref, w2_ref, b2_ref, w3_ref, b3_ref,
                   out_ref, agg_ref, v_scr, c_scr):
    b = pl.program_id(0)
    ne = ne_ref[...]

    @pl.when(b == 0)
    def _prep():
        wa = wa_ref[...]
        ww_top = ww_ref[0:_A, :]
        ww_bot = ww_ref[_A:2 * _A, :]
        v_scr[...] = jax.lax.dot_general(
            ww_bot, wa, (((0,), (1,)), ((), ())),
            preferred_element_type=jnp.float32)
        u = jnp.dot(wa, ww_top, preferred_element_type=jnp.float32)
        c_all = jnp.dot(now_ref[...], u, preferred_element_type=jnp.float32)
        c_all = c_all + jnp.dot(ba_ref[...], ww_top, preferred_element_type=jnp.float32)
        c_all = c_all + jnp.dot(ba_ref[...], ww_bot, preferred_element_type=jnp.float32)
        c_all = c_all + bw_ref[...]
        for s in range(_STEPS):
            c_scr[pl.ds(s * 8, _R), :] = c_all[s * _R:(s + 1) * _R, :]

    scores = jax.lax.dot_general(
        v_scr[...], ne, (((1,), (1,)), ((), ())),
        preferred_element_type=jnp.float32)
    c_blk = c_scr[pl.ds(pl.multiple_of(b * 8, 8), _R), :]

    aggs = []
    for r in range(_R):
        s = scores[:, r * _L:(r + 1) * _L] + c_blk[r:r + 1, :]
        s = jnp.where(s >= 0.0, s, 0.2 * s)
        m = jnp.max(s, axis=1, keepdims=True)
        e = jnp.exp(s - m)
        alpha = e / jnp.sum(e, axis=1, keepdims=True)
        aggs.append(jnp.dot(alpha, ne[r * _L:(r + 1) * _L, :],
                            preferred_element_type=jnp.float32))
    agg_ref[pl.ds(pl.multiple_of(b * 8, 8), _R), :] = jnp.concatenate(aggs, axis=0)

    @pl.when(b == _STEPS - 1)
    def _mlp():
        st = state_ref[...]
        ag = jnp.concatenate(
            [agg_ref[pl.ds(s * 8, _R), :] for s in range(_STEPS)], axis=0)
        h = jnp.dot(st, w1_ref[0:2 * _D, :], preferred_element_type=jnp.float32)
        h = h + jnp.dot(ag, w1_ref[2 * _D:3 * _D, :], preferred_element_type=jnp.float32)
        h = jax.nn.relu(h + b1_ref[...])
        h = jax.nn.relu(jnp.dot(h, w2_ref[...], preferred_element_type=jnp.float32) + b2_ref[...])
        logits = jnp.dot(h, w3_ref[...], preferred_element_type=jnp.float32) + b3_ref[...]
        z = logits - jnp.max(logits, axis=1, keepdims=True)
        ez = jnp.exp(z)
        out_ref[...] = ez / jnp.sum(ez, axis=1, keepdims=True)


def kernel(state, now_embedding, neighbour_embeddings, Wa, ba, Ww, bw, W1, b1, W2, b2, W3, b3):
    ne_flat = neighbour_embeddings.reshape(_B * _L, _D)
    ba2 = ba.reshape(1, _A)
    bw2 = bw.reshape(1, 1)
    b12 = b1.reshape(1, _H1)
    b22 = b2.reshape(1, _H2)
    b32 = b3.reshape(1, _ACT)

    const = lambda *_: (0, 0)
    in_specs = [
            pl.BlockSpec((_B, 2 * _D), const),
            pl.BlockSpec((_B, _D), const),
            pl.BlockSpec((_R * _L, _D), lambda b: (b, 0)),
            pl.BlockSpec((_D, _A), const),
            pl.BlockSpec((1, _A), const),
            pl.BlockSpec((2 * _A, 1), const),
            pl.BlockSpec((1, 1), const),
            pl.BlockSpec((3 * _D, _H1), const),
            pl.BlockSpec((1, _H1), const),
            pl.BlockSpec((_H1, _H2), const),
            pl.BlockSpec((1, _H2), const),
            pl.BlockSpec((_H2, _ACT), const),
            pl.BlockSpec((1, _ACT), const),
    ]
    return pl.pallas_call(
        _attn_mlp_body,
        grid=(_STEPS,),
        in_specs=in_specs,
        out_specs=pl.BlockSpec((_B, _ACT), const),
        out_shape=jax.ShapeDtypeStruct((_B, _ACT), jnp.float32),
        scratch_shapes=[pltpu.VMEM((_STEPS * 8, _D), jnp.float32),
                        pltpu.VMEM((1, _D), jnp.float32),
                        pltpu.VMEM((_STEPS * 8, 1), jnp.float32)],
    )(state, now_embedding, ne_flat, Wa, ba2, Ww, bw2,
      W1, b12, W2, b22, W3, b32)
